# baseline (speedup 1.0000x reference)
import functools
import jax
import jax.numpy as jnp
from jax import lax
from jax.experimental import pallas as pl
from jax.experimental.pallas import tpu as pltpu
from jax.experimental.pallas import tpu_sc as plsc

B = 16384
D = 64
L = 16


@jax.jit
def _bprmf_gather(user, pos_item, neg_item, embed_user, embed_item):
    info = plsc.get_sparse_core_info()
    nc, ns = info.num_cores, info.num_subcores
    nw = nc * ns
    bpw = B // nw
    mesh = plsc.VectorSubcoreMesh(core_axis_name="c", subcore_axis_name="s")

    @functools.partial(
        pl.kernel,
        mesh=mesh,
        out_type=(
            jax.ShapeDtypeStruct((B, D), jnp.float32),
            jax.ShapeDtypeStruct((B, D), jnp.float32),
            jax.ShapeDtypeStruct((B, D), jnp.float32),
        ),
        scratch_types=[
            pltpu.VMEM((bpw,), jnp.int32),
            pltpu.SemaphoreType.DMA,
        ],
    )
    def k(user_hbm, pos_hbm, neg_hbm, eu_hbm, ei_hbm,
          out_u, out_p, out_n, idx_v, sem):
        wid = lax.axis_index("s") * nc + lax.axis_index("c")
        base = wid * bpw

        def one_table(idx_hbm, tab_hbm, out_hbm):
            pltpu.sync_copy(idx_hbm.at[pl.ds(base, bpw)], idx_v)

            def group_body(g, _):
                v16 = idx_v[pl.ds(g * L, L)]
                for jj in range(L):
                    r = v16[jj]
                    pltpu.async_copy(
                        tab_hbm.at[r], out_hbm.at[base + g * L + jj], sem)
                return _
            lax.fori_loop(0, bpw // L, group_body, 0)

        one_table(user_hbm, eu_hbm, out_u)
        one_table(pos_hbm, ei_hbm, out_p)
        one_table(neg_hbm, ei_hbm, out_n)

        pltpu.make_async_copy(
            out_u.at[pl.ds(base, bpw)], out_u.at[pl.ds(base, bpw)], sem).wait()
        pltpu.make_async_copy(
            out_p.at[pl.ds(base, bpw)], out_p.at[pl.ds(base, bpw)], sem).wait()
        pltpu.make_async_copy(
            out_n.at[pl.ds(base, bpw)], out_n.at[pl.ds(base, bpw)], sem).wait()

    return k(user, pos_item, neg_item, embed_user, embed_item)


def kernel(user, pos_item, neg_item, embed_user, embed_item):
    return _bprmf_gather(user, pos_item, neg_item, embed_user, embed_item)


# speedup vs baseline: 2.0082x; 2.0082x over previous
import functools
import jax
import jax.numpy as jnp
from jax import lax
from jax.experimental import pallas as pl
from jax.experimental.pallas import tpu as pltpu
from jax.experimental.pallas import tpu_sc as plsc

B = 16384
D = 64
L = 16


@jax.jit
def _bprmf_gather(user, pos_item, neg_item, embed_user, embed_item):
    info = plsc.get_sparse_core_info()
    nc, ns = info.num_cores, info.num_subcores
    nw = nc * ns
    bpw = B // nw
    mesh = plsc.VectorSubcoreMesh(core_axis_name="c", subcore_axis_name="s")

    @functools.partial(
        pl.kernel,
        mesh=mesh,
        out_type=(
            jax.ShapeDtypeStruct((B, D), jnp.float32),
            jax.ShapeDtypeStruct((B, D), jnp.float32),
            jax.ShapeDtypeStruct((B, D), jnp.float32),
        ),
        scratch_types=[
            pltpu.VMEM((bpw,), jnp.int32),
            pltpu.VMEM((bpw, D), jnp.float32),
            pltpu.SemaphoreType.DMA,
        ],
    )
    def k(user_hbm, pos_hbm, neg_hbm, eu_hbm, ei_hbm,
          out_u, out_p, out_n, idx_v, rows_v, sem):
        wid = lax.axis_index("s") * nc + lax.axis_index("c")
        base = wid * bpw

        def one_table(idx_hbm, tab_hbm, out_hbm):
            pltpu.sync_copy(idx_hbm.at[pl.ds(base, bpw)], idx_v)

            def group_body(g, _):
                v16 = idx_v[pl.ds(g * L, L)]
                for jj in range(L):
                    r = v16[jj]
                    pltpu.async_copy(
                        tab_hbm.at[r], rows_v.at[g * L + jj], sem)
                return _
            lax.fori_loop(0, bpw // L, group_body, 0)

            pltpu.make_async_copy(
                tab_hbm.at[pl.ds(0, bpw)], rows_v, sem).wait()
            pltpu.sync_copy(rows_v, out_hbm.at[pl.ds(base, bpw)])

        one_table(user_hbm, eu_hbm, out_u)
        one_table(pos_hbm, ei_hbm, out_p)
        one_table(neg_hbm, ei_hbm, out_n)

    return k(user, pos_item, neg_item, embed_user, embed_item)


def kernel(user, pos_item, neg_item, embed_user, embed_item):
    return _bprmf_gather(user, pos_item, neg_item, embed_user, embed_item)
